# baseline (speedup 1.0000x reference)
import functools

import jax
import jax.numpy as jnp
from jax import lax
from jax.experimental import pallas as pl
from jax.experimental.pallas import tpu as pltpu
from jax.experimental.pallas import tpu_sc as plsc

NUM_CORES = 2
NUM_SUBCORES = 16
NUM_WORKERS = NUM_CORES * NUM_SUBCORES
LANES = 16

BATCH = 16384
EMB_DIM = 64
NUM_FULL_TILES = 7812
TAIL_BASE = NUM_FULL_TILES * 128
CAPL = 8
ID_CHUNK = 1024
RING = 4
ROWS_PER_WORKER = BATCH // NUM_WORKERS
CHUNK = 128
NUM_CHUNKS = ROWS_PER_WORKER // CHUNK
BLOCKS_PER_CHUNK = CHUNK // LANES

_SC_PARAMS = pltpu.CompilerParams(
    needs_layout_passes=False, use_tc_tiling_on_sc=True)
_SC_MESH = plsc.VectorSubcoreMesh(
    core_axis_name="c", subcore_axis_name="s",
    num_cores=NUM_CORES, num_subcores=NUM_SUBCORES)

_NT_BIG = 248
_NT_SMALL = 244
_BUCKET_TILES = _NT_BIG + 1


def _gather_body(ids_hbm, tab_hbm, tail_hbm, g_hbm,
                 ids_v, cnts_v, bkts_v, buf0, buf1, buf2, buf3, stage_v,
                 semW0, semW1, semW2, semW3, semT,
                 s0, s1, s2, s3, s4, s5, s6, s7,
                 s8, s9, s10, s11, s12, s13, s14, s15):
    bufs = [buf0, buf1, buf2, buf3]
    wsems = [semW0, semW1, semW2, semW3]
    lane_sems = [s0, s1, s2, s3, s4, s5, s6, s7,
                 s8, s9, s10, s11, s12, s13, s14, s15]
    wid = lax.axis_index("s") * NUM_CORES + lax.axis_index("c")
    t0 = jnp.where(wid < 1, 0, _NT_BIG + _NT_SMALL * (wid - 1))
    nt = jnp.where(wid < 1, _NT_BIG, _NT_SMALL)
    is_last = (wid == NUM_WORKERS - 1).astype(jnp.int32)

    iota = lax.iota(jnp.int32, LANES)
    zeros16 = jnp.zeros((LANES,), jnp.int32)

    def zero_counts(i, c):
        cnts_v[pl.ds(i * LANES, LANES)] = zeros16
        return c

    lax.fori_loop(0, _BUCKET_TILES, zero_counts, 0)

    def bucket_chunk(ci, c):
        kbase = ci * ID_CHUNK
        pltpu.sync_copy(ids_hbm.at[pl.ds(kbase, ID_CHUNK)], ids_v)

        def bucket_step(j, c2):
            idv = ids_v[pl.ds(j * LANES, LANES)]
            t = lax.shift_right_logical(idv, 7)
            mine = (t >= t0) & (t < t0 + nt + is_last)
            tloc = jnp.where(mine, t - t0, 0)
            k16 = kbase + j * LANES + iota
            pay = lax.shift_left(k16, 7) | (idv & 127)
            cidx = tloc * LANES + iota
            cnt = plsc.load_gather(cnts_v, [cidx], mask=mine)
            cnt = jnp.where(mine, cnt, CAPL)
            ok = mine & (cnt < CAPL)
            slotaddr = cidx * CAPL + jnp.where(ok, cnt, 0)
            plsc.store_scatter(bkts_v, [slotaddr], pay, mask=ok)
            plsc.addupdate_scatter(
                cnts_v, [cidx], jnp.ones((LANES,), jnp.int32), mask=ok)
            return c2

        lax.fori_loop(0, ID_CHUNK // LANES, bucket_step, 0)
        return c

    lax.fori_loop(0, BATCH // ID_CHUNK, bucket_chunk, 0)

    for L in range(LANES):
        pltpu.async_copy(g_hbm.at[0], stage_v.at[L], lane_sems[L])

    def fire(tile, buf, sem):
        off = pl.multiple_of(tile * 128, 128)
        pltpu.async_copy(tab_hbm.at[:, pl.ds(off, 128)], buf, sem)

    def wait_buf(buf, sem):
        pltpu.make_async_copy(tab_hbm.at[:, pl.ds(0, 128)], buf, sem).wait()

    def extract_tile(tloc, win):
        cl16 = cnts_v[pl.ds(tloc * LANES, LANES)]
        for L in range(LANES):
            cnt = cl16[L]
            base = (tloc * LANES + L) * CAPL

            def body(e, c, base=base, L=L):
                pltpu.make_async_copy(
                    g_hbm.at[0], stage_v.at[L], lane_sems[L]).wait()
                ent = bkts_v[pl.ds(base + e, 16)][0]
                k = lax.shift_right_logical(ent, 7)
                lanev = jnp.full((LANES,), 0, jnp.int32) + (ent & 127)
                for q in range(4):
                    colq = plsc.load_gather(win, [iota + q * LANES, lanev])
                    stage_v[L, pl.ds(q * LANES, LANES)] = colq
                pltpu.async_copy(stage_v.at[L], g_hbm.at[k], lane_sems[L])
                return c

            lax.fori_loop(0, cnt, body, 0)

    for s in range(RING):
        fire(t0 + s, bufs[s], wsems[s])

    def ring_step(p, c):
        for s in range(RING):
            g = t0 + RING * p + s
            wait_buf(bufs[s], wsems[s])
            extract_tile(g - t0, bufs[s])
            nxt = jnp.where(g + RING < t0 + nt, g + RING, t0)
            fire(nxt, bufs[s], wsems[s])
        return c

    lax.fori_loop(0, nt // RING, ring_step, 0)
    for s in range(RING):
        wait_buf(bufs[s], wsems[s])

    cl16 = cnts_v[pl.ds(nt * LANES, LANES)]
    for L in range(LANES):
        cnt = cl16[L]
        base = (nt * LANES + L) * CAPL

        def tbody(e, c, base=base, L=L):
            pltpu.make_async_copy(
                g_hbm.at[0], stage_v.at[L], lane_sems[L]).wait()
            ent = bkts_v[pl.ds(base + e, 16)][0]
            k = lax.shift_right_logical(ent, 7)
            row = ent & 127
            pltpu.async_copy(tail_hbm.at[row], stage_v.at[L], semT).wait()
            pltpu.async_copy(stage_v.at[L], g_hbm.at[k], lane_sems[L])
            return c

        lax.fori_loop(0, cnt, tbody, 0)

    for L in range(LANES):
        pltpu.make_async_copy(
            g_hbm.at[0], stage_v.at[L], lane_sems[L]).wait()


_gather_kernel = functools.partial(
    pl.kernel,
    out_type=jax.ShapeDtypeStruct((BATCH, EMB_DIM), jnp.float32),
    mesh=_SC_MESH,
    compiler_params=_SC_PARAMS,
    scratch_types=[
        pltpu.VMEM((ID_CHUNK,), jnp.int32),
        pltpu.VMEM((_BUCKET_TILES * LANES + 16,), jnp.int32),
        pltpu.VMEM((_BUCKET_TILES * LANES * CAPL + 16,), jnp.int32),
        pltpu.VMEM((EMB_DIM, 128), jnp.float32),
        pltpu.VMEM((EMB_DIM, 128), jnp.float32),
        pltpu.VMEM((EMB_DIM, 128), jnp.float32),
        pltpu.VMEM((EMB_DIM, 128), jnp.float32),
        pltpu.VMEM((LANES, EMB_DIM), jnp.float32),
        pltpu.SemaphoreType.DMA,
        pltpu.SemaphoreType.DMA,
        pltpu.SemaphoreType.DMA,
        pltpu.SemaphoreType.DMA,
        pltpu.SemaphoreType.DMA,
    ] + [pltpu.SemaphoreType.DMA] * LANES,
)(_gather_body)


def _compute_body(ug_hbm, vg_hbm, wb_hbm, b_hbm, out_hbm,
                  urows_v, irows_v, b_v, wb_v, out_v):
    wid = lax.axis_index("s") * NUM_CORES + lax.axis_index("c")
    base = wid * ROWS_PER_WORKER

    pltpu.sync_copy(wb_hbm, wb_v)
    pltpu.sync_copy(b_hbm, b_v)
    bvec = b_v[:]
    iota = lax.iota(jnp.int32, LANES)

    def block(blk, carry):
        rows = (blk % BLOCKS_PER_CHUNK) * LANES + iota
        acc = jnp.zeros((LANES,), jnp.float32)
        for d in range(EMB_DIM):
            col = jnp.full((LANES,), d, jnp.int32)
            ucol = plsc.load_gather(urows_v, [rows, col])
            vcol = plsc.load_gather(irows_v, [rows, col])
            acc = acc + ucol * vcol * wb_v[pl.ds(d * LANES, LANES)]
        logits = acc + bvec
        out_v[pl.ds(blk * LANES, LANES)] = 1.0 / (1.0 + jnp.exp(-logits))
        return carry

    for ch in range(NUM_CHUNKS):
        pltpu.sync_copy(ug_hbm.at[pl.ds(base + ch * CHUNK, CHUNK)], urows_v)
        pltpu.sync_copy(vg_hbm.at[pl.ds(base + ch * CHUNK, CHUNK)], irows_v)
        lax.fori_loop(ch * BLOCKS_PER_CHUNK, (ch + 1) * BLOCKS_PER_CHUNK,
                      block, 0)

    pltpu.sync_copy(out_v, out_hbm.at[pl.ds(base, ROWS_PER_WORKER)])


_compute_kernel = functools.partial(
    pl.kernel,
    out_type=jax.ShapeDtypeStruct((BATCH,), jnp.float32),
    mesh=_SC_MESH,
    compiler_params=_SC_PARAMS,
    scratch_types=[
        pltpu.VMEM((CHUNK, EMB_DIM), jnp.float32),
        pltpu.VMEM((CHUNK, EMB_DIM), jnp.float32),
        pltpu.VMEM((LANES,), jnp.float32),
        pltpu.VMEM((EMB_DIM * LANES,), jnp.float32),
        pltpu.VMEM((ROWS_PER_WORKER,), jnp.float32),
    ],
)(_compute_body)


@jax.jit
def kernel(user_ids, item_ids, user_table, item_table, W, b):
    uids = user_ids.astype(jnp.int32)
    iids = item_ids.astype(jnp.int32)
    ut_t = user_table.T
    it_t = item_table.T
    tail_u = lax.slice(user_table, (TAIL_BASE, 0), (1000000, EMB_DIM))
    tail_v = lax.slice(item_table, (TAIL_BASE, 0), (1000000, EMB_DIM))
    wb = jnp.broadcast_to(W.reshape(EMB_DIM, 1).astype(jnp.float32),
                          (EMB_DIM, LANES)).reshape(EMB_DIM * LANES)
    b16 = jnp.broadcast_to(b.astype(jnp.float32), (LANES,))
    ug = _gather_kernel(uids, ut_t, tail_u)
    vg = _gather_kernel(iids, it_t, tail_v)
    out = _compute_kernel(ug, vg, wb, b16)
    return out.reshape(BATCH, 1)


# speedup vs baseline: 1.7313x; 1.7313x over previous
import functools

import jax
import jax.numpy as jnp
from jax import lax
from jax.experimental import pallas as pl
from jax.experimental.pallas import tpu as pltpu
from jax.experimental.pallas import tpu_sc as plsc

NUM_CORES = 2
NUM_SUBCORES = 16
NUM_WORKERS = NUM_CORES * NUM_SUBCORES
LANES = 16

BATCH = 16384
EMB_DIM = 64
NUM_FULL_TILES = 7812
TAIL_BASE = NUM_FULL_TILES * 128
CAPL = 8
ID_CHUNK = 1024
RING = 4
ROWS_PER_WORKER = BATCH // NUM_WORKERS
CHUNK = 128
NUM_CHUNKS = ROWS_PER_WORKER // CHUNK
BLOCKS_PER_CHUNK = CHUNK // LANES

_SC_PARAMS = pltpu.CompilerParams(
    needs_layout_passes=False, use_tc_tiling_on_sc=True)
_SC_MESH = plsc.VectorSubcoreMesh(
    core_axis_name="c", subcore_axis_name="s",
    num_cores=NUM_CORES, num_subcores=NUM_SUBCORES)

_NT_BIG = 248
_NT_SMALL = 244
_BUCKET_TILES = _NT_BIG + 1


def _gather_body(ids_hbm, tab_hbm, tail_hbm, g_hbm,
                 ids_v, cnts_v, bkts_v, buf0, buf1, buf2, buf3, stage_v,
                 semW0, semW1, semW2, semW3, semT,
                 s0, s1, s2, s3, s4, s5, s6, s7,
                 s8, s9, s10, s11, s12, s13, s14, s15):
    bufs = [buf0, buf1, buf2, buf3]
    wsems = [semW0, semW1, semW2, semW3]
    lane_sems = [s0, s1, s2, s3, s4, s5, s6, s7,
                 s8, s9, s10, s11, s12, s13, s14, s15]
    wid = lax.axis_index("s") * NUM_CORES + lax.axis_index("c")
    t0 = jnp.where(wid < 1, 0, _NT_BIG + _NT_SMALL * (wid - 1))
    nt = jnp.where(wid < 1, _NT_BIG, _NT_SMALL)
    is_last = (wid == NUM_WORKERS - 1).astype(jnp.int32)

    iota = lax.iota(jnp.int32, LANES)
    zeros16 = jnp.zeros((LANES,), jnp.int32)

    def zero_counts(i, c):
        cnts_v[pl.ds(i * LANES, LANES)] = zeros16
        return c

    lax.fori_loop(0, _BUCKET_TILES, zero_counts, 0)

    def bucket_chunk(ci, c):
        kbase = ci * ID_CHUNK
        pltpu.sync_copy(ids_hbm.at[pl.ds(kbase, ID_CHUNK)], ids_v)

        def bucket_step(j, c2):
            idv = ids_v[pl.ds(j * LANES, LANES)]
            t = lax.shift_right_logical(idv, 7)
            mine = (t >= t0) & (t < t0 + nt + is_last)
            tloc = jnp.where(mine, t - t0, 0)
            k16 = kbase + j * LANES + iota
            pay = lax.shift_left(k16, 7) | (idv & 127)
            cidx = tloc * LANES + iota
            cnt = plsc.load_gather(cnts_v, [cidx], mask=mine)
            cnt = jnp.where(mine, cnt, CAPL)
            ok = mine & (cnt < CAPL)
            slotaddr = cidx * CAPL + jnp.where(ok, cnt, 0)
            plsc.store_scatter(bkts_v, [slotaddr], pay, mask=ok)
            plsc.addupdate_scatter(
                cnts_v, [cidx], jnp.ones((LANES,), jnp.int32), mask=ok)
            return c2

        lax.fori_loop(0, ID_CHUNK // LANES, bucket_step, 0)
        return c

    lax.fori_loop(0, BATCH // ID_CHUNK, bucket_chunk, 0)

    for L in range(LANES):
        pltpu.async_copy(g_hbm.at[0], stage_v.at[L], lane_sems[L])

    def fire(tile, buf, sem):
        off = pl.multiple_of(tile * 128, 128)
        pltpu.async_copy(tab_hbm.at[:, pl.ds(off, 128)], buf, sem)

    def wait_buf(buf, sem):
        pltpu.make_async_copy(tab_hbm.at[:, pl.ds(0, 128)], buf, sem).wait()

    def extract_tile(tloc, win):
        cl16 = cnts_v[pl.ds(tloc * LANES, LANES)]
        for L in range(LANES):
            cnt = cl16[L]
            base = (tloc * LANES + L) * CAPL

            def body(e, c, base=base, L=L):
                pltpu.make_async_copy(
                    g_hbm.at[0], stage_v.at[L], lane_sems[L]).wait()
                ent = bkts_v[pl.ds(base + e, 16)][0]
                k = lax.shift_right_logical(ent, 7)
                lanev = jnp.full((LANES,), 0, jnp.int32) + (ent & 127)
                for q in range(4):
                    colq = plsc.load_gather(win, [iota + q * LANES, lanev])
                    stage_v[L, pl.ds(q * LANES, LANES)] = colq
                pltpu.async_copy(stage_v.at[L], g_hbm.at[k], lane_sems[L])
                return c

            lax.fori_loop(0, cnt, body, 0)

    for s in range(RING):
        fire(t0 + s, bufs[s], wsems[s])

    def ring_step(p, c):
        for s in range(RING):
            g = t0 + RING * p + s
            wait_buf(bufs[s], wsems[s])
            extract_tile(g - t0, bufs[s])
            nxt = jnp.where(g + RING < t0 + nt, g + RING, t0)
            fire(nxt, bufs[s], wsems[s])
        return c

    lax.fori_loop(0, nt // RING, ring_step, 0)
    for s in range(RING):
        wait_buf(bufs[s], wsems[s])

    cl16 = cnts_v[pl.ds(nt * LANES, LANES)]
    for L in range(LANES):
        cnt = cl16[L]
        base = (nt * LANES + L) * CAPL

        def tbody(e, c, base=base, L=L):
            pltpu.make_async_copy(
                g_hbm.at[0], stage_v.at[L], lane_sems[L]).wait()
            ent = bkts_v[pl.ds(base + e, 16)][0]
            k = lax.shift_right_logical(ent, 7)
            row = ent & 127
            pltpu.async_copy(tail_hbm.at[row], stage_v.at[L], semT).wait()
            pltpu.async_copy(stage_v.at[L], g_hbm.at[k], lane_sems[L])
            return c

        lax.fori_loop(0, cnt, tbody, 0)

    for L in range(LANES):
        pltpu.make_async_copy(
            g_hbm.at[0], stage_v.at[L], lane_sems[L]).wait()


_gather_kernel = functools.partial(
    pl.kernel,
    out_type=jax.ShapeDtypeStruct((BATCH, EMB_DIM), jnp.float32),
    mesh=_SC_MESH,
    compiler_params=_SC_PARAMS,
    scratch_types=[
        pltpu.VMEM((ID_CHUNK,), jnp.int32),
        pltpu.VMEM((_BUCKET_TILES * LANES + 16,), jnp.int32),
        pltpu.VMEM((_BUCKET_TILES * LANES * CAPL + 16,), jnp.int32),
        pltpu.VMEM((EMB_DIM, 128), jnp.float32),
        pltpu.VMEM((EMB_DIM, 128), jnp.float32),
        pltpu.VMEM((EMB_DIM, 128), jnp.float32),
        pltpu.VMEM((EMB_DIM, 128), jnp.float32),
        pltpu.VMEM((LANES, EMB_DIM), jnp.float32),
        pltpu.SemaphoreType.DMA,
        pltpu.SemaphoreType.DMA,
        pltpu.SemaphoreType.DMA,
        pltpu.SemaphoreType.DMA,
        pltpu.SemaphoreType.DMA,
    ] + [pltpu.SemaphoreType.DMA] * LANES,
)(_gather_body)


def _item_body(iids_hbm, itab_hbm, ug_hbm, wb_hbm, b_hbm, out_hbm,
               iidx_v, urows_v, irows_v, b_v, wb_v, out_v, isem):
    wid = lax.axis_index("s") * NUM_CORES + lax.axis_index("c")
    base = wid * ROWS_PER_WORKER

    pltpu.sync_copy(iids_hbm.at[pl.ds(base, ROWS_PER_WORKER)], iidx_v)
    pltpu.sync_copy(wb_hbm, wb_v)
    pltpu.sync_copy(b_hbm, b_v)

    bvec = b_v[:]
    iota = lax.iota(jnp.int32, LANES)

    def fire(blk, carry):
        ivec = iidx_v[pl.ds(blk * LANES, LANES)]
        dst = (blk % BLOCKS_PER_CHUNK) * LANES
        for j in range(LANES):
            pltpu.async_copy(
                itab_hbm.at[ivec[j]], irows_v.at[dst + j], isem)
        return carry

    def compute(blk, carry):
        rows = (blk % BLOCKS_PER_CHUNK) * LANES + iota
        acc = jnp.zeros((LANES,), jnp.float32)
        for d in range(EMB_DIM):
            col = jnp.full((LANES,), d, jnp.int32)
            ucol = plsc.load_gather(urows_v, [rows, col])
            vcol = plsc.load_gather(irows_v, [rows, col])
            acc = acc + ucol * vcol * wb_v[pl.ds(d * LANES, LANES)]
        logits = acc + bvec
        out_v[pl.ds(blk * LANES, LANES)] = 1.0 / (1.0 + jnp.exp(-logits))
        return carry

    for ch in range(NUM_CHUNKS):
        lax.fori_loop(ch * BLOCKS_PER_CHUNK, (ch + 1) * BLOCKS_PER_CHUNK,
                      fire, 0)
        pltpu.sync_copy(
            ug_hbm.at[pl.ds(base + ch * CHUNK, CHUNK)], urows_v)
        pltpu.make_async_copy(
            itab_hbm.at[pl.ds(0, CHUNK)], irows_v, isem).wait()
        lax.fori_loop(ch * BLOCKS_PER_CHUNK, (ch + 1) * BLOCKS_PER_CHUNK,
                      compute, 0)

    pltpu.sync_copy(out_v, out_hbm.at[pl.ds(base, ROWS_PER_WORKER)])


_item_kernel = functools.partial(
    pl.kernel,
    out_type=jax.ShapeDtypeStruct((BATCH,), jnp.float32),
    mesh=_SC_MESH,
    compiler_params=_SC_PARAMS,
    scratch_types=[
        pltpu.VMEM((ROWS_PER_WORKER,), jnp.int32),
        pltpu.VMEM((CHUNK, EMB_DIM), jnp.float32),
        pltpu.VMEM((CHUNK, EMB_DIM), jnp.float32),
        pltpu.VMEM((LANES,), jnp.float32),
        pltpu.VMEM((EMB_DIM * LANES,), jnp.float32),
        pltpu.VMEM((ROWS_PER_WORKER,), jnp.float32),
        pltpu.SemaphoreType.DMA,
    ],
)(_item_body)


@jax.jit
def kernel(user_ids, item_ids, user_table, item_table, W, b):
    uids = user_ids.astype(jnp.int32)
    iids = item_ids.astype(jnp.int32)
    ut_t = user_table.T
    tail_u = lax.slice(user_table, (TAIL_BASE, 0), (1000000, EMB_DIM))
    wb = jnp.broadcast_to(W.reshape(EMB_DIM, 1).astype(jnp.float32),
                          (EMB_DIM, LANES)).reshape(EMB_DIM * LANES)
    b16 = jnp.broadcast_to(b.astype(jnp.float32), (LANES,))
    ug = _gather_kernel(uids, ut_t, tail_u)
    out = _item_kernel(iids, item_table, ug, wb, b16)
    return out.reshape(BATCH, 1)
